# Initial kernel scaffold; baseline (speedup 1.0000x reference)
import jax
import jax.numpy as jnp
from jax.experimental import pallas as pl


def kernel(x, W_enc, b_enc, W_dec, b_dec):
    raise NotImplementedError("write your pallas kernel here")


# speedup vs baseline: 8.2368x; 8.2368x over previous
import functools

import jax
import jax.numpy as jnp
from jax.experimental import pallas as pl
from jax.experimental.pallas import tpu as pltpu

_F32_INF_BITS = 0x7F800000


def _encode_topk_body(x_ref, w_ref, b_ref, z_ref, zs_ref, *, n_l, bl, k, n_search):
    l = pl.program_id(1)
    zt = jax.lax.dot_general(
        x_ref[...], w_ref[...],
        dimension_numbers=(((1,), (1,)), ((), ())),
        preferred_element_type=jnp.float32,
    )
    zt = jnp.maximum(zt + b_ref[0], 0.0)
    zs_ref[:, pl.ds(l * bl, bl)] = zt

    @pl.when(l == n_l - 1)
    def _finalize():
        bt = zs_ref.shape[0]

        def body(_, carry):
            lo, hi = carry
            mid = lo + ((hi - lo) >> 1)
            zb = jax.lax.bitcast_convert_type(zs_ref[...], jnp.int32)
            cnt = jnp.sum((zb >= mid).astype(jnp.int32), axis=1, keepdims=True)
            ge = cnt >= k
            return jnp.where(ge, mid, lo), jnp.where(ge, hi, mid)

        lo0 = jnp.zeros((bt, 1), jnp.int32)
        hi0 = jnp.full((bt, 1), _F32_INF_BITS, jnp.int32)
        lo, _ = jax.lax.fori_loop(0, n_search, body, (lo0, hi0))
        z = zs_ref[...]
        zb = jax.lax.bitcast_convert_type(z, jnp.int32)
        z_ref[...] = jnp.where(zb >= lo, z, 0.0)


def _decode_body(z_ref, w_ref, b_ref, o_ref, acc_ref, *, n_k):
    kk = pl.program_id(1)

    @pl.when(kk == 0)
    def _init():
        acc_ref[...] = jnp.broadcast_to(b_ref[...], acc_ref.shape)

    acc_ref[...] += jax.lax.dot_general(
        z_ref[...].astype(jnp.bfloat16), w_ref[...],
        dimension_numbers=(((1,), (0,)), ((), ())),
        preferred_element_type=jnp.float32,
    )

    @pl.when(kk == n_k - 1)
    def _emit():
        o_ref[...] = acc_ref[...]


@functools.partial(jax.jit, static_argnames=("topk",))
def _run(x, W_enc, b_enc, W_dec, b_dec, topk=64):
    n_tok, d_in = x.shape
    d_lat = W_enc.shape[0]

    bt = 128 if n_tok % 128 == 0 else n_tok
    bl = 1024 if d_lat % 1024 == 0 else d_lat
    n_t, n_l = n_tok // bt, d_lat // bl
    b_enc3 = b_enc.reshape(n_l, 1, bl)
    x_bf = x.astype(jnp.bfloat16)
    w_enc_bf = W_enc.astype(jnp.bfloat16)

    z_masked = pl.pallas_call(
        functools.partial(_encode_topk_body, n_l=n_l, bl=bl, k=topk, n_search=31),
        grid=(n_t, n_l),
        in_specs=[
            pl.BlockSpec((bt, d_in), lambda t, l: (t, 0)),
            pl.BlockSpec((bl, d_in), lambda t, l: (l, 0)),
            pl.BlockSpec((1, 1, bl), lambda t, l: (l, 0, 0)),
        ],
        out_specs=pl.BlockSpec((bt, d_lat), lambda t, l: (t, 0)),
        out_shape=jax.ShapeDtypeStruct((n_tok, d_lat), jnp.float32),
        scratch_shapes=[pltpu.VMEM((bt, d_lat), jnp.float32)],
    )(x_bf, w_enc_bf, b_enc3)

    w_dec_t = W_dec.T.astype(jnp.bfloat16)
    b_dec2 = b_dec.reshape(1, d_in)
    bt2 = 512 if n_tok % 512 == 0 else n_tok
    bk = 2048 if d_lat % 2048 == 0 else d_lat
    n_t2, n_k = n_tok // bt2, d_lat // bk

    recon = pl.pallas_call(
        functools.partial(_decode_body, n_k=n_k),
        grid=(n_t2, n_k),
        in_specs=[
            pl.BlockSpec((bt2, bk), lambda t, kk: (t, kk)),
            pl.BlockSpec((bk, d_in), lambda t, kk: (kk, 0)),
            pl.BlockSpec((1, d_in), lambda t, kk: (0, 0)),
        ],
        out_specs=pl.BlockSpec((bt2, d_in), lambda t, kk: (t, 0)),
        out_shape=jax.ShapeDtypeStruct((n_tok, d_in), jnp.float32),
        scratch_shapes=[pltpu.VMEM((bt2, d_in), jnp.float32)],
    )(z_masked, w_dec_t, b_dec2)

    return recon, z_masked


def kernel(x, W_enc, b_enc, W_dec, b_dec):
    return _run(x, W_enc, b_enc, W_dec, b_dec)


# speedup vs baseline: 10.5127x; 1.2763x over previous
import functools

import jax
import jax.numpy as jnp
from jax.experimental import pallas as pl
from jax.experimental.pallas import tpu as pltpu

_F32_INF_BITS = 0x7F800000


def _search_iters(zs, lo_ref, hi_ref, k, n_iter):
    zb = jax.lax.bitcast_convert_type(zs, jnp.int32)

    def body(_, carry):
        lo, hi = carry
        mid = lo + ((hi - lo) >> 1)
        cnt = jnp.sum((zb >= mid).astype(jnp.int32), axis=1, keepdims=True)
        ge = cnt >= k
        return jnp.where(ge, mid, lo), jnp.where(ge, hi, mid)

    lo, hi = jax.lax.fori_loop(0, n_iter, body, (lo_ref[...], hi_ref[...]))
    lo_ref[...] = lo
    hi_ref[...] = hi


def _encode_body(x_ref, w_ref, b_ref, z_ref, thr_ref, zs_ref, lo_ref, hi_ref,
                 *, n_l, bl, k):
    t = pl.program_id(0)
    l = pl.program_id(1)
    zt = jax.lax.dot_general(
        x_ref[...], w_ref[...],
        dimension_numbers=(((1,), (1,)), ((), ())),
        preferred_element_type=jnp.float32,
    )
    zt = jnp.maximum(zt + b_ref[0], 0.0)
    z_ref[...] = zt
    zs_ref[t % 2, :, pl.ds(l * bl, bl)] = zt

    bt = zs_ref.shape[1]

    @pl.when(t > 0)
    def _search():
        @pl.when(l == 0)
        def _init():
            lo_ref[...] = jnp.zeros((bt, 1), jnp.int32)
            hi_ref[...] = jnp.full((bt, 1), _F32_INF_BITS, jnp.int32)

        avail = max(n_l - 1, 1)
        base, extra = 31 // avail, 31 % avail

        @pl.when(l < avail)
        def _iters():
            _search_iters(zs_ref[(t - 1) % 2], lo_ref, hi_ref, k, base)

        @pl.when(l == avail - 1)
        def _last_iter_and_emit():
            if extra:
                _search_iters(zs_ref[(t - 1) % 2], lo_ref, hi_ref, k, extra)
            thr_ref[...] = lo_ref[...]

    @pl.when((t == pl.num_programs(0) - 1) & (l == n_l - 1))
    def _tail():
        lo_ref[...] = jnp.zeros((bt, 1), jnp.int32)
        hi_ref[...] = jnp.full((bt, 1), _F32_INF_BITS, jnp.int32)
        _search_iters(zs_ref[t % 2], lo_ref, hi_ref, k, 31)
        thr_ref[...] = lo_ref[...]


def _decode_body(z_ref, thr_ref, w_ref, b_ref, o_ref, zm_ref, acc_ref, *, n_k):
    kk = pl.program_id(1)

    @pl.when(kk == 0)
    def _init():
        acc_ref[...] = jnp.broadcast_to(b_ref[...], acc_ref.shape)

    z = z_ref[...]
    zb = jax.lax.bitcast_convert_type(z, jnp.int32)
    zm = jnp.where(zb >= thr_ref[...], z, 0.0)
    zm_ref[...] = zm
    acc_ref[...] += jax.lax.dot_general(
        zm.astype(jnp.bfloat16), w_ref[...],
        dimension_numbers=(((1,), (0,)), ((), ())),
        preferred_element_type=jnp.float32,
    )

    @pl.when(kk == n_k - 1)
    def _emit():
        o_ref[...] = acc_ref[...]


@functools.partial(jax.jit, static_argnames=("topk",))
def _run(x, W_enc, b_enc, W_dec, b_dec, topk=64):
    n_tok, d_in = x.shape
    d_lat = W_enc.shape[0]

    bt = 256 if n_tok % 256 == 0 else n_tok
    bl = 512 if d_lat % 512 == 0 else d_lat
    n_t, n_l = n_tok // bt, d_lat // bl
    b_enc3 = b_enc.reshape(n_l, 1, bl)
    x_bf = x.astype(jnp.bfloat16)
    w_enc_bf = W_enc.astype(jnp.bfloat16)

    z_pre, thr = pl.pallas_call(
        functools.partial(_encode_body, n_l=n_l, bl=bl, k=topk),
        grid=(n_t, n_l),
        in_specs=[
            pl.BlockSpec((bt, d_in), lambda t, l: (t, 0)),
            pl.BlockSpec((bl, d_in), lambda t, l: (l, 0)),
            pl.BlockSpec((1, 1, bl), lambda t, l: (l, 0, 0)),
        ],
        out_specs=[
            pl.BlockSpec((bt, bl), lambda t, l: (t, l)),
            pl.BlockSpec(
                (bt, 1),
                lambda t, l: (
                    jnp.where((t == n_t - 1) & (l == n_l - 1),
                              t, jnp.maximum(t - 1, 0)), 0)),
        ],
        out_shape=[
            jax.ShapeDtypeStruct((n_tok, d_lat), jnp.float32),
            jax.ShapeDtypeStruct((n_tok, 1), jnp.int32),
        ],
        scratch_shapes=[
            pltpu.VMEM((2, bt, d_lat), jnp.float32),
            pltpu.VMEM((bt, 1), jnp.int32),
            pltpu.VMEM((bt, 1), jnp.int32),
        ],
    )(x_bf, w_enc_bf, b_enc3)

    w_dec_t = W_dec.T.astype(jnp.bfloat16)
    b_dec2 = b_dec.reshape(1, d_in)
    bt2 = 512 if n_tok % 512 == 0 else n_tok
    bk = 2048 if d_lat % 2048 == 0 else d_lat
    n_t2, n_k = n_tok // bt2, d_lat // bk

    recon, z_masked = pl.pallas_call(
        functools.partial(_decode_body, n_k=n_k),
        grid=(n_t2, n_k),
        in_specs=[
            pl.BlockSpec((bt2, bk), lambda t, kk: (t, kk)),
            pl.BlockSpec((bt2, 1), lambda t, kk: (t, 0)),
            pl.BlockSpec((bk, d_in), lambda t, kk: (kk, 0)),
            pl.BlockSpec((1, d_in), lambda t, kk: (0, 0)),
        ],
        out_specs=[
            pl.BlockSpec((bt2, d_in), lambda t, kk: (t, 0)),
            pl.BlockSpec((bt2, bk), lambda t, kk: (t, kk)),
        ],
        out_shape=[
            jax.ShapeDtypeStruct((n_tok, d_in), jnp.float32),
            jax.ShapeDtypeStruct((n_tok, d_lat), jnp.float32),
        ],
        scratch_shapes=[pltpu.VMEM((bt2, d_in), jnp.float32)],
    )(z_pre, thr, w_dec_t, b_dec2)

    return recon, z_masked


def kernel(x, W_enc, b_enc, W_dec, b_dec):
    return _run(x, W_enc, b_enc, W_dec, b_dec)


# speedup vs baseline: 10.9117x; 1.0379x over previous
import functools

import jax
import jax.numpy as jnp
from jax.experimental import pallas as pl
from jax.experimental.pallas import tpu as pltpu

_F32_INF_BITS = 0x7F800000


def _search_iters(zs, lo_ref, hi_ref, k, n_iter):
    zb = jax.lax.bitcast_convert_type(zs, jnp.int32)

    def body(_, carry):
        lo, hi = carry
        mid = lo + ((hi - lo) >> 1)
        cnt = jnp.sum((zb >= mid[None]).astype(jnp.int32), axis=(0, 2))[:, None]
        ge = cnt >= k
        eq = cnt == k
        lo = jnp.where(ge, mid, lo)
        hi = jnp.where(eq, mid + 1, jnp.where(ge, hi, mid))
        return lo, hi

    lo, hi = jax.lax.fori_loop(0, n_iter, body, (lo_ref[...], hi_ref[...]))
    lo_ref[...] = lo
    hi_ref[...] = hi


def _encode_body(x_ref, w_ref, b_ref, z_ref, thr_ref, zs_ref, lo_ref, hi_ref,
                 *, n_l, bl, k):
    t = pl.program_id(0)
    l = pl.program_id(1)
    zt = jax.lax.dot_general(
        x_ref[...], w_ref[...],
        dimension_numbers=(((1,), (1,)), ((), ())),
        preferred_element_type=jnp.float32,
    )
    zt = jnp.maximum(zt + b_ref[0], 0.0)
    z_ref[...] = zt
    zs_ref[t % 2, l] = zt

    bt = zs_ref.shape[2]

    @pl.when(t > 0)
    def _search():
        @pl.when(l == 0)
        def _init():
            lo_ref[...] = jnp.zeros((bt, 1), jnp.int32)
            hi_ref[...] = jnp.full((bt, 1), _F32_INF_BITS, jnp.int32)

        avail = max(n_l - 1, 1)
        base, extra = 31 // avail, 31 % avail

        @pl.when(l < avail)
        def _iters():
            @pl.when(jnp.any(hi_ref[...] - lo_ref[...] > 1))
            def _go():
                n_iter = jnp.where(l == avail - 1, base + extra, base)
                _search_iters(zs_ref[(t - 1) % 2], lo_ref, hi_ref, k, n_iter)

        @pl.when(l == avail - 1)
        def _emit():
            thr_ref[...] = lo_ref[...]

    @pl.when((t == pl.num_programs(0) - 1) & (l == n_l - 1))
    def _tail():
        lo_ref[...] = jnp.zeros((bt, 1), jnp.int32)
        hi_ref[...] = jnp.full((bt, 1), _F32_INF_BITS, jnp.int32)
        _search_iters(zs_ref[t % 2], lo_ref, hi_ref, k, 31)
        thr_ref[...] = lo_ref[...]


def _decode_body(z_ref, thr_ref, w_ref, b_ref, o_ref, zm_ref, acc_ref, *, n_k):
    kk = pl.program_id(1)

    @pl.when(kk == 0)
    def _init():
        acc_ref[...] = jnp.broadcast_to(b_ref[...], acc_ref.shape)

    z = z_ref[...]
    zb = jax.lax.bitcast_convert_type(z, jnp.int32)
    zm = jnp.where(zb >= thr_ref[...], z, 0.0)
    zm_ref[...] = zm
    acc_ref[...] += jax.lax.dot_general(
        zm.astype(jnp.bfloat16), w_ref[...],
        dimension_numbers=(((1,), (0,)), ((), ())),
        preferred_element_type=jnp.float32,
    )

    @pl.when(kk == n_k - 1)
    def _emit():
        o_ref[...] = acc_ref[...]


@functools.partial(jax.jit, static_argnames=("topk",))
def _run(x, W_enc, b_enc, W_dec, b_dec, topk=64):
    n_tok, d_in = x.shape
    d_lat = W_enc.shape[0]

    bt = 256 if n_tok % 256 == 0 else n_tok
    bl = 512 if d_lat % 512 == 0 else d_lat
    n_t, n_l = n_tok // bt, d_lat // bl
    b_enc3 = b_enc.reshape(n_l, 1, bl)
    x_bf = x.astype(jnp.bfloat16)
    w_enc_bf = W_enc.astype(jnp.bfloat16)

    z_pre, thr = pl.pallas_call(
        functools.partial(_encode_body, n_l=n_l, bl=bl, k=topk),
        grid=(n_t, n_l),
        in_specs=[
            pl.BlockSpec((bt, d_in), lambda t, l: (t, 0)),
            pl.BlockSpec((bl, d_in), lambda t, l: (l, 0)),
            pl.BlockSpec((1, 1, bl), lambda t, l: (l, 0, 0)),
        ],
        out_specs=[
            pl.BlockSpec((bt, bl), lambda t, l: (t, l)),
            pl.BlockSpec(
                (bt, 1),
                lambda t, l: (
                    jnp.where((t == n_t - 1) & (l == n_l - 1),
                              t, jnp.maximum(t - 1, 0)), 0)),
        ],
        out_shape=[
            jax.ShapeDtypeStruct((n_tok, d_lat), jnp.float32),
            jax.ShapeDtypeStruct((n_tok, 1), jnp.int32),
        ],
        scratch_shapes=[
            pltpu.VMEM((2, n_l, bt, bl), jnp.float32),
            pltpu.VMEM((bt, 1), jnp.int32),
            pltpu.VMEM((bt, 1), jnp.int32),
        ],
    )(x_bf, w_enc_bf, b_enc3)

    w_dec_t = W_dec.T.astype(jnp.bfloat16)
    b_dec2 = b_dec.reshape(1, d_in)
    bt2 = 512 if n_tok % 512 == 0 else n_tok
    bk = 2048 if d_lat % 2048 == 0 else d_lat
    n_t2, n_k = n_tok // bt2, d_lat // bk

    recon, z_masked = pl.pallas_call(
        functools.partial(_decode_body, n_k=n_k),
        grid=(n_t2, n_k),
        in_specs=[
            pl.BlockSpec((bt2, bk), lambda t, kk: (t, kk)),
            pl.BlockSpec((bt2, 1), lambda t, kk: (t, 0)),
            pl.BlockSpec((bk, d_in), lambda t, kk: (kk, 0)),
            pl.BlockSpec((1, d_in), lambda t, kk: (0, 0)),
        ],
        out_specs=[
            pl.BlockSpec((bt2, d_in), lambda t, kk: (t, 0)),
            pl.BlockSpec((bt2, bk), lambda t, kk: (t, kk)),
        ],
        out_shape=[
            jax.ShapeDtypeStruct((n_tok, d_in), jnp.float32),
            jax.ShapeDtypeStruct((n_tok, d_lat), jnp.float32),
        ],
        scratch_shapes=[pltpu.VMEM((bt2, d_in), jnp.float32)],
    )(z_pre, thr, w_dec_t, b_dec2)

    return recon, z_masked


def kernel(x, W_enc, b_enc, W_dec, b_dec):
    return _run(x, W_enc, b_enc, W_dec, b_dec)


# speedup vs baseline: 11.1419x; 1.0211x over previous
import functools

import jax
import jax.numpy as jnp
from jax.experimental import pallas as pl
from jax.experimental.pallas import tpu as pltpu

_F32_INF_BITS = 0x7F800000


def _search_iters(zs_ref, base, lo_ref, hi_ref, k, n_iter):
    n_l = zs_ref.shape[0] // 2
    d_lat = n_l * zs_ref.shape[2]

    def body(_, carry):
        lo, hi = carry
        mid = lo + ((hi - lo) >> 1)
        acc = jnp.zeros(zs_ref.shape[1:], jnp.int32)
        for lp in range(n_l):
            zb = jax.lax.bitcast_convert_type(zs_ref[base + lp], jnp.int32)
            acc = acc + jax.lax.shift_right_arithmetic(zb - mid, 31)
        cnt = d_lat + jnp.sum(acc, axis=1, keepdims=True)
        ge = cnt >= k
        eq = cnt == k
        lo = jnp.where(ge, mid, lo)
        hi = jnp.where(eq, mid + 1, jnp.where(ge, hi, mid))
        return lo, hi

    lo, hi = jax.lax.fori_loop(0, n_iter, body, (lo_ref[...], hi_ref[...]))
    lo_ref[...] = lo
    hi_ref[...] = hi


def _encode_body(x_ref, w_ref, b_ref, z_ref, thr_ref, zs_ref, lo_ref, hi_ref,
                 *, n_l, bl, k):
    t = pl.program_id(0)
    l = pl.program_id(1)
    zt = jax.lax.dot_general(
        x_ref[...], w_ref[...],
        dimension_numbers=(((1,), (1,)), ((), ())),
        preferred_element_type=jnp.float32,
    )
    zt = jnp.maximum(zt + b_ref[0], 0.0)
    z_ref[...] = zt
    zs_ref[(t % 2) * n_l + l] = zt

    bt = zs_ref.shape[1]

    @pl.when(t > 0)
    def _search():
        @pl.when(l == 0)
        def _init():
            lo_ref[...] = jnp.zeros((bt, 1), jnp.int32)
            hi_ref[...] = jnp.full((bt, 1), _F32_INF_BITS, jnp.int32)

        avail = max(n_l - 1, 1)
        base, extra = 31 // avail, 31 % avail

        @pl.when(l < avail)
        def _iters():
            @pl.when(jnp.any(hi_ref[...] - lo_ref[...] > 1))
            def _go():
                n_iter = jnp.where(l == avail - 1, base + extra, base)
                _search_iters(zs_ref, ((t - 1) % 2) * n_l, lo_ref, hi_ref,
                              k, n_iter)

        @pl.when(l == avail - 1)
        def _emit():
            thr_ref[...] = lo_ref[...]

    @pl.when((t == pl.num_programs(0) - 1) & (l == n_l - 1))
    def _tail():
        lo_ref[...] = jnp.zeros((bt, 1), jnp.int32)
        hi_ref[...] = jnp.full((bt, 1), _F32_INF_BITS, jnp.int32)
        _search_iters(zs_ref, (t % 2) * n_l, lo_ref, hi_ref, k, 31)
        thr_ref[...] = lo_ref[...]


def _decode_body(z_ref, thr_ref, w_ref, b_ref, o_ref, zm_ref, acc_ref, *, n_k):
    kk = pl.program_id(1)

    @pl.when(kk == 0)
    def _init():
        acc_ref[...] = jnp.broadcast_to(b_ref[...], acc_ref.shape)

    z = z_ref[...]
    zb = jax.lax.bitcast_convert_type(z, jnp.int32)
    zm = jnp.where(zb >= thr_ref[...], z, 0.0)
    zm_ref[...] = zm
    acc_ref[...] += jax.lax.dot_general(
        zm.astype(jnp.bfloat16), w_ref[...],
        dimension_numbers=(((1,), (0,)), ((), ())),
        preferred_element_type=jnp.float32,
    )

    @pl.when(kk == n_k - 1)
    def _emit():
        o_ref[...] = acc_ref[...]


@functools.partial(jax.jit, static_argnames=("topk",))
def _run(x, W_enc, b_enc, W_dec, b_dec, topk=64):
    n_tok, d_in = x.shape
    d_lat = W_enc.shape[0]

    bt = 256 if n_tok % 256 == 0 else n_tok
    bl = 512 if d_lat % 512 == 0 else d_lat
    n_t, n_l = n_tok // bt, d_lat // bl
    b_enc3 = b_enc.reshape(n_l, 1, bl)
    x_bf = x.astype(jnp.bfloat16)
    w_enc_bf = W_enc.astype(jnp.bfloat16)

    z_pre, thr = pl.pallas_call(
        functools.partial(_encode_body, n_l=n_l, bl=bl, k=topk),
        grid=(n_t, n_l),
        in_specs=[
            pl.BlockSpec((bt, d_in), lambda t, l: (t, 0)),
            pl.BlockSpec((bl, d_in), lambda t, l: (l, 0)),
            pl.BlockSpec((1, 1, bl), lambda t, l: (l, 0, 0)),
        ],
        out_specs=[
            pl.BlockSpec((bt, bl), lambda t, l: (t, l)),
            pl.BlockSpec(
                (bt, 1),
                lambda t, l: (
                    jnp.where((t == n_t - 1) & (l == n_l - 1),
                              t, jnp.maximum(t - 1, 0)), 0)),
        ],
        out_shape=[
            jax.ShapeDtypeStruct((n_tok, d_lat), jnp.float32),
            jax.ShapeDtypeStruct((n_tok, 1), jnp.int32),
        ],
        scratch_shapes=[
            pltpu.VMEM((2 * n_l, bt, bl), jnp.float32),
            pltpu.VMEM((bt, 1), jnp.int32),
            pltpu.VMEM((bt, 1), jnp.int32),
        ],
    )(x_bf, w_enc_bf, b_enc3)

    w_dec_t = W_dec.T.astype(jnp.bfloat16)
    b_dec2 = b_dec.reshape(1, d_in)
    bt2 = 512 if n_tok % 512 == 0 else n_tok
    bk = 2048 if d_lat % 2048 == 0 else d_lat
    n_t2, n_k = n_tok // bt2, d_lat // bk

    recon, z_masked = pl.pallas_call(
        functools.partial(_decode_body, n_k=n_k),
        grid=(n_t2, n_k),
        in_specs=[
            pl.BlockSpec((bt2, bk), lambda t, kk: (t, kk)),
            pl.BlockSpec((bt2, 1), lambda t, kk: (t, 0)),
            pl.BlockSpec((bk, d_in), lambda t, kk: (kk, 0)),
            pl.BlockSpec((1, d_in), lambda t, kk: (0, 0)),
        ],
        out_specs=[
            pl.BlockSpec((bt2, d_in), lambda t, kk: (t, 0)),
            pl.BlockSpec((bt2, bk), lambda t, kk: (t, kk)),
        ],
        out_shape=[
            jax.ShapeDtypeStruct((n_tok, d_in), jnp.float32),
            jax.ShapeDtypeStruct((n_tok, d_lat), jnp.float32),
        ],
        scratch_shapes=[pltpu.VMEM((bt2, d_in), jnp.float32)],
    )(z_pre, thr, w_dec_t, b_dec2)

    return recon, z_masked


def kernel(x, W_enc, b_enc, W_dec, b_dec):
    return _run(x, W_enc, b_enc, W_dec, b_dec)
